# baseline (speedup 1.0000x reference)
import functools

import jax
import jax.numpy as jnp
from jax import lax
from jax.experimental import pallas as pl
from jax.experimental.pallas import tpu as pltpu
from jax.experimental.pallas import tpu_sc as plsc

F32 = jnp.float32
BF = jnp.bfloat16

N_NODES = 10000
N_EDGES = 160000
B = 500
NPG = 20

NE_PAD = 163840
NODE_BLK = 2000
EDGE_BLK = 2048
N_NODE_BLKS = N_NODES // NODE_BLK
N_EDGE_BLKS = NE_PAD // EDGE_BLK
GPB = NODE_BLK // NPG

NC = 2
NS = 16
NW = NC * NS
GCH = 128
CH = 128
SPS = NE_PAD // CH // NS
NN_PAD = 10240
B_PAD = 512
SRC_PAD = 10100
DST_PAD = 10200


def _dot(a, b):
    return jnp.dot(a, b, preferred_element_type=F32)


def _sdot(s, x):
    xh = x.astype(BF).astype(F32)
    xl = x - xh
    return _dot(s, xh) + _dot(s, xl)


def _global_body(u, w1, b1, w2, b2, w3, b3, uh_o, uhb_o):
    h = jax.nn.relu(_dot(u[...], w1[...]) + b1[...])
    h = jax.nn.relu(_dot(h, w2[...]) + b2[...])
    uh = _dot(h, w3[...]) + b3[...]
    uh_o[...] = uh
    uhb_o[...] = uh.astype(BF)


def _node_embed_body(xc, w1, b1, w2, b2, w3, b3, xhb_o):
    h = jax.nn.relu(_dot(xc[...], w1[...]) + b1[...])
    h = jax.nn.relu(_dot(h, w2[...]) + b2[...])
    xh = _dot(h, w3[...]) + b3[...]
    xhb_o[...] = xh.astype(BF)


def _edge1_body(ea, w1, b1, w2, b2, w3, b3, ehb_o):
    h = jax.nn.relu(_dot(ea[...], w1[...]) + b1[...])
    h = jax.nn.relu(_dot(h, w2[...]) + b2[...])
    eh = _dot(h, w3[...]) + b3[...]
    ehb_o[...] = eh.astype(BF)


def _edge2_body(ehb, ga, gp, gu, w1, pb1, w2, b2, eh_o):
    e_in = jnp.concatenate([ehb[...], ga[...], gp[...], gu[...]], axis=1)
    z = jax.nn.relu(_dot(e_in, w1[...]) + pb1[...])
    eh_o[...] = _dot(z.astype(BF), w2[...]) + b2[...]


def _node_upd_body(xhb, agge, urep, w1, bn1, w2, bn2, aggn_o):
    n_in = jnp.concatenate([xhb[...], agge[...].astype(BF), urep[...]],
                           axis=1)
    h = jax.nn.relu(_dot(n_in, w1[...]) + bn1[...])
    x2 = _dot(h.astype(BF), w2[...]) + bn2[...]
    rows = lax.broadcasted_iota(jnp.int32, (GPB, NODE_BLK), 1)
    cols = lax.broadcasted_iota(jnp.int32, (GPB, NODE_BLK), 0)
    st = (rows // NPG == cols).astype(F32)
    aggn_o[...] = _sdot(st, x2)[None]


def _global_upd_body(uh, aggn, aggg, w1, bu1, w2, bu2,
                     hw1, hb1, hw2, hb2, q_o):
    u_in = jnp.concatenate([uh[...].astype(BF), aggn[...].astype(BF),
                            aggg[...].astype(BF)], axis=1)
    h = jax.nn.relu(_dot(u_in, w1[...]) + bu1[...])
    u2 = _dot(h.astype(BF), w2[...]) + bu2[...]
    hh = jax.nn.relu(_dot(u2.astype(BF), hw1[...]) + hb1[...])
    q_o[...] = _dot(hh.astype(BF), hw2[...]) + hb2[...]


def _full(shape):
    n = len(shape)
    return pl.BlockSpec(shape, lambda *_: (0,) * n)


def _global_embed(u, w1, b1, w2, b2, w3, b3):
    return pl.pallas_call(
        _global_body,
        in_specs=[_full((B_PAD, 16)),
                  _full((16, 512)), _full((1, 512)),
                  _full((512, 512)), _full((1, 512)),
                  _full((512, 512)), _full((1, 512))],
        out_specs=[_full((B_PAD, 512)), _full((B_PAD, 512))],
        out_shape=[jax.ShapeDtypeStruct((B_PAD, 512), F32),
                   jax.ShapeDtypeStruct((B_PAD, 512), BF)],
    )(u, w1, b1, w2, b2, w3, b3)


def _node_embed(xc, w1, b1, w2, b2, w3, b3):
    return pl.pallas_call(
        _node_embed_body,
        grid=(N_NODE_BLKS,),
        in_specs=[pl.BlockSpec((NODE_BLK, 21), lambda i: (i, 0)),
                  _full((21, 512)), _full((1, 512)),
                  _full((512, 512)), _full((1, 512)),
                  _full((512, 512)), _full((1, 512))],
        out_specs=pl.BlockSpec((NODE_BLK, 512), lambda i: (i, 0)),
        out_shape=jax.ShapeDtypeStruct((NN_PAD, 512), BF),
    )(xc, w1, b1, w2, b2, w3, b3)


def _edge1(ea, w1, b1, w2, b2, w3, b3):
    return pl.pallas_call(
        _edge1_body,
        grid=(N_EDGE_BLKS,),
        in_specs=[pl.BlockSpec((EDGE_BLK, 4), lambda i: (i, 0)),
                  _full((4, 256)), _full((1, 256)),
                  _full((256, 256)), _full((1, 256)),
                  _full((256, 256)), _full((1, 256))],
        out_specs=pl.BlockSpec((EDGE_BLK, 256), lambda i: (i, 0)),
        out_shape=jax.ShapeDtypeStruct((NE_PAD, 256), BF),
    )(ea, w1, b1, w2, b2, w3, b3)


def _edge2(ehb, ga, gp, gu, w1, pb1, w2, b2):
    def espec(n):
        return pl.BlockSpec((EDGE_BLK, n), lambda i: (i, 0))
    return pl.pallas_call(
        _edge2_body,
        grid=(N_EDGE_BLKS,),
        in_specs=[espec(256), espec(512), espec(512), espec(512),
                  _full((1792, 256)), _full((1, 256)),
                  _full((256, 256)), _full((1, 256))],
        out_specs=espec(256),
        out_shape=jax.ShapeDtypeStruct((NE_PAD, 256), F32),
    )(ehb, ga, gp, gu, w1, pb1, w2, b2)


def _node_upd(xhb, agge, urep, w1, bn1, w2, bn2):
    return pl.pallas_call(
        _node_upd_body,
        grid=(N_NODE_BLKS,),
        in_specs=[pl.BlockSpec((NODE_BLK, 512), lambda i: (i, 0)),
                  pl.BlockSpec((NODE_BLK, 256), lambda i: (i, 0)),
                  pl.BlockSpec((NODE_BLK, 512), lambda i: (i, 0)),
                  _full((1280, 512)), _full((1, 512)),
                  _full((512, 512)), _full((1, 512))],
        out_specs=pl.BlockSpec((1, GPB, 512), lambda i: (i, 0, 0)),
        out_shape=jax.ShapeDtypeStruct((N_NODE_BLKS, GPB, 512), F32),
    )(xhb, agge, urep, w1, bn1, w2, bn2)


def _global_upd(uh, aggn, aggg, w1, bu1, w2, bu2, hw1, hb1, hw2, hb2):
    return pl.pallas_call(
        _global_upd_body,
        in_specs=[_full((B_PAD, 512)), _full((B_PAD, 512)),
                  _full((B_PAD, 256)),
                  _full((1280, 512)), _full((1, 512)),
                  _full((512, 512)), _full((1, 512)),
                  _full((512, 256)), _full((1, 256)),
                  _full((256, 1)), _full((1, 1))],
        out_specs=_full((B_PAD, 1)),
        out_shape=jax.ShapeDtypeStruct((B_PAD, 1), F32),
    )(uh, aggn, aggg, w1, bu1, w2, bu2, hw1, hb1, hw2, hb2)


def _sc_gather(xh_tab, u_tab, src2, dst2):
    mesh = plsc.VectorSubcoreMesh(core_axis_name="c", subcore_axis_name="s")
    cpw = NE_PAD // GCH // NW

    @functools.partial(
        pl.kernel, mesh=mesh,
        out_type=[jax.ShapeDtypeStruct((NE_PAD, 256), jnp.int32),
                  jax.ShapeDtypeStruct((NE_PAD, 256), jnp.int32),
                  jax.ShapeDtypeStruct((NE_PAD, 256), jnp.int32)],
        scratch_types=[pltpu.VMEM((1, GCH), jnp.int32),
                       pltpu.VMEM((1, GCH), jnp.int32),
                       pltpu.VMEM((1, GCH), jnp.int32),
                       pltpu.VMEM((GCH, 256), jnp.int32),
                       pltpu.VMEM((GCH, 256), jnp.int32),
                       pltpu.VMEM((GCH, 256), jnp.int32),
                       pltpu.SemaphoreType.DMA,
                       pltpu.SemaphoreType.DMA,
                       pltpu.SemaphoreType.DMA,
                       pltpu.SemaphoreType.DMA],
    )
    def k(xh_hbm, u_hbm, src_hbm, dst_hbm, ga_hbm, gp_hbm, gu_hbm,
          si_v, di_v, gi_v, ra_v, rp_v, ru_v, sga, sgp, sgu, semw):
        wid = lax.axis_index("s") * NC + lax.axis_index("c")

        def body(j, carry):
            chunk = j * NW + wid
            off = chunk * GCH
            pltpu.sync_copy(src_hbm.at[pl.ds(chunk, 1)], si_v)
            pltpu.sync_copy(dst_hbm.at[pl.ds(chunk, 1)], di_v)
            for kk in range(GCH // 16):
                sl = pl.ds(kk * 16, 16)
                gi_v[0, sl] = lax.div(si_v[0, sl], NPG)
            ha = pltpu.async_copy(xh_hbm.at[si_v.at[0]], ra_v, sga)
            hp = pltpu.async_copy(xh_hbm.at[di_v.at[0]], rp_v, sgp)
            hu = pltpu.async_copy(u_hbm.at[gi_v.at[0]], ru_v, sgu)
            ha.wait()
            wa = pltpu.async_copy(ra_v, ga_hbm.at[pl.ds(off, GCH)], semw)
            hp.wait()
            wp = pltpu.async_copy(rp_v, gp_hbm.at[pl.ds(off, GCH)], semw)
            hu.wait()
            wu = pltpu.async_copy(ru_v, gu_hbm.at[pl.ds(off, GCH)], semw)
            wa.wait()
            wp.wait()
            wu.wait()
            return carry

        lax.fori_loop(0, cpw, body, 0)

    return k(xh_tab, u_tab, src2, dst2)


def _sc_scatter(eh, src2, dst2):
    mesh = plsc.VectorSubcoreMesh(core_axis_name="c", subcore_axis_name="s")
    epb = NN_PAD // NS // CH
    gpb = B_PAD // NS

    @functools.partial(
        pl.kernel, mesh=mesh,
        out_type=[jax.ShapeDtypeStruct((NN_PAD, 256), F32),
                  jax.ShapeDtypeStruct((B_PAD, 256), F32)],
        scratch_types=[pltpu.VMEM((2, CH, 128), F32),
                       pltpu.VMEM((2, CH), jnp.int32),
                       pltpu.VMEM((2, CH), jnp.int32),
                       pltpu.VMEM((gpb, 128), F32),
                       pltpu.VMEM_SHARED((NN_PAD, 128), F32),
                       pltpu.VMEM_SHARED((B_PAD, 128), F32),
                       pltpu.SemaphoreType.DMA,
                       pltpu.SemaphoreType.DMA],
    )
    def k(eh_hbm, src_hbm, dst_hbm, agge_hbm, aggg_hbm,
          vb, di_v, gi_v, zbuf, acce, accg, sl0, sl1):
        c = lax.axis_index("c")
        s = lax.axis_index("s")
        base = s * SPS * CH
        brow = s * SPS

        def zrow(r, carry):
            for kk in range(128 // 16):
                zbuf[r, pl.ds(kk * 16, 16)] = jnp.zeros((16,), F32)
            return carry

        lax.fori_loop(0, gpb, zrow, 0)
        for kk in range(NN_PAD // NS // gpb):
            pltpu.sync_copy(zbuf, acce.at[pl.ds((s * 20 + kk) * gpb, gpb)])
        pltpu.sync_copy(zbuf, accg.at[pl.ds(s * gpb, gpb)])
        plsc.subcore_barrier()

        sls = (sl0, sl1)

        def body(jj, carry):
            hs = []
            for t in range(2):
                j = jj * 2 + t
                off = base + j * CH
                hs.append(pltpu.async_copy(
                    eh_hbm.at[pl.ds(off, CH), pl.ds(c * 128, 128)],
                    vb.at[t], sls[t]))
                pltpu.sync_copy(dst_hbm.at[brow + j], di_v.at[t])
                pltpu.sync_copy(src_hbm.at[brow + j], gi_v.at[t])
            for t in range(2):
                for kk in range(CH // 16):
                    sl = pl.ds(kk * 16, 16)
                    gi_v[t, sl] = lax.div(gi_v[t, sl], NPG)
                hs[t].wait()
                pltpu.sync_copy(vb.at[t], acce.at[di_v.at[t]], add=True)
                pltpu.sync_copy(vb.at[t], accg.at[gi_v.at[t]], add=True)
            return carry

        lax.fori_loop(0, SPS // 2, body, 0)
        plsc.subcore_barrier()

        for kk in range(epb):
            r = (s * epb + kk) * CH
            pltpu.sync_copy(acce.at[pl.ds(r, CH)],
                            agge_hbm.at[pl.ds(r, CH), pl.ds(c * 128, 128)])
        r2 = s * gpb
        pltpu.sync_copy(accg.at[pl.ds(r2, gpb)],
                        aggg_hbm.at[pl.ds(r2, gpb), pl.ds(c * 128, 128)])

    return k(eh, src2, dst2)


def _branch(p, xc, edge_attr, src2, dst2, u_pad):
    def w(name, i):
        return p[name][i]['W']

    def bb(name, i):
        return p[name][i]['b'].reshape(1, -1)

    peW1 = p['gnn']['phi_e'][0]['W'].astype(BF)
    peb1 = p['gnn']['phi_e'][0]['b'].reshape(1, -1)
    peW2 = p['gnn']['phi_e'][1]['W'].astype(BF)
    peb2 = p['gnn']['phi_e'][1]['b'].reshape(1, -1)
    pnW1 = p['gnn']['phi_n'][0]['W'].astype(BF)
    pnb1 = p['gnn']['phi_n'][0]['b'].reshape(1, -1)
    pnW2 = p['gnn']['phi_n'][1]['W'].astype(BF)
    pnb2 = p['gnn']['phi_n'][1]['b'].reshape(1, -1)
    puW1 = p['gnn']['phi_u'][0]['W'].astype(BF)
    pub1 = p['gnn']['phi_u'][0]['b'].reshape(1, -1)
    puW2 = p['gnn']['phi_u'][1]['W'].astype(BF)
    pub2 = p['gnn']['phi_u'][1]['b'].reshape(1, -1)
    hW1 = p['head'][0]['W'].astype(BF)
    hb1 = bb('head', 0)
    hW2 = p['head'][1]['W'].astype(BF)
    hb2 = bb('head', 1)

    uh, uhb = _global_embed(u_pad, w('g_emb', 0), bb('g_emb', 0),
                            w('g_emb', 1), bb('g_emb', 1),
                            w('g_emb', 2), bb('g_emb', 2))
    xhb = _node_embed(xc, w('n_emb', 0), bb('n_emb', 0),
                      w('n_emb', 1), bb('n_emb', 1),
                      w('n_emb', 2), bb('n_emb', 2))
    urep = jnp.repeat(uhb[:B], NPG, axis=0)
    ehb = _edge1(edge_attr, w('e_emb', 0), bb('e_emb', 0),
                 w('e_emb', 1), bb('e_emb', 1),
                 w('e_emb', 2), bb('e_emb', 2))
    xh32 = lax.bitcast_convert_type(
        xhb.reshape(NN_PAD, 256, 2), jnp.int32)
    u32 = lax.bitcast_convert_type(
        uhb.reshape(B_PAD, 256, 2), jnp.int32)
    ga, gp, gu = [
        lax.bitcast_convert_type(g, BF).reshape(NE_PAD, 512)
        for g in _sc_gather(xh32, u32, src2, dst2)]
    eh = _edge2(ehb, ga, gp, gu, peW1, peb1, peW2, peb2)
    agge, aggg = _sc_scatter(eh, src2, dst2)
    aggn = _node_upd(xhb, agge, urep, pnW1, pnb1, pnW2, pnb2)
    aggn_pad = jnp.pad(aggn.reshape(B, 512), ((0, B_PAD - B), (0, 0)))
    q = _global_upd(uh, aggn_pad, aggg, puW1, pub1, puW2, pub2,
                    hW1, hb1, hW2, hb2)
    return q[:B]


def kernel(x, edge_index, edge_attr, u, batch, action, params):
    del batch

    bsz, adim = action.shape
    apd = adim // 2
    robot = action.reshape(bsz, 2, apd)
    full = jnp.concatenate(
        [robot, jnp.zeros((bsz, NPG - 2, apd), dtype=x.dtype)], axis=1)
    xc = jnp.concatenate([x, full.reshape(-1, apd)], axis=1)
    npad = NE_PAD - N_EDGES
    src = jnp.concatenate(
        [edge_index[0], jnp.full((npad,), SRC_PAD, jnp.int32)])
    dst = jnp.concatenate(
        [edge_index[1], jnp.full((npad,), DST_PAD, jnp.int32)])
    edge_attr = jnp.pad(edge_attr, ((0, npad), (0, 0)))
    u_pad = jnp.pad(u, ((0, B_PAD - B), (0, 0)))
    src2 = src.reshape(NE_PAD // GCH, GCH)
    dst2 = dst.reshape(NE_PAD // GCH, GCH)

    q1 = _branch(params['branch1'], xc, edge_attr, src2, dst2, u_pad)
    q2 = _branch(params['branch2'], xc, edge_attr, src2, dst2, u_pad)
    return (q1, q2)


# speedup vs baseline: 3.0335x; 3.0335x over previous
import functools

import jax
import jax.numpy as jnp
from jax import lax
from jax.experimental import pallas as pl
from jax.experimental.pallas import tpu as pltpu
from jax.experimental.pallas import tpu_sc as plsc

F32 = jnp.float32
BF = jnp.bfloat16

N_NODES = 10000
N_EDGES = 160000
B = 500
NPG = 20

NE_PAD = 163840
NODE_BLK = 2000
EDGE_BLK = 1024
N_NODE_BLKS = N_NODES // NODE_BLK
N_EDGE_BLKS = NE_PAD // EDGE_BLK
GPB = NODE_BLK // NPG

NC = 2
NS = 16
NW = NC * NS
GCH = 64
CH = 128
SPS = NE_PAD // CH // NS
NN_PAD = 10240
B_PAD = 512
SRC_PAD = 10100
DST_PAD = 10200


def _dot(a, b):
    return jnp.dot(a, b, preferred_element_type=F32)


def _sdot(s, x):
    xh = x.astype(BF).astype(F32)
    xl = x - xh
    return _dot(s, xh) + _dot(s, xl)


def _global_body(u, w1, b1, w2, b2, w3, b3, uh_o, uhb_o):
    h = jax.nn.relu(_dot(u[...], w1[...]) + b1[...])
    h = jax.nn.relu(_dot(h, w2[...]) + b2[...])
    uh = _dot(h, w3[...]) + b3[...]
    uh_o[...] = uh
    uhb_o[...] = uh.astype(BF).astype(F32)


def _node_embed_body(xc, w1, b1, w2, b2, w3, b3, xhb_o):
    h = jax.nn.relu(_dot(xc[...], w1[...]) + b1[...])
    h = jax.nn.relu(_dot(h, w2[...]) + b2[...])
    xh = _dot(h, w3[...]) + b3[...]
    xhb_o[...] = xh.astype(BF).astype(F32)


def _edge1_body(ea, w1, b1, w2, b2, w3, b3, ehb_o):
    h = jax.nn.relu(_dot(ea[...], w1[...]) + b1[...])
    h = jax.nn.relu(_dot(h, w2[...]) + b2[...])
    eh = _dot(h, w3[...]) + b3[...]
    ehb_o[...] = eh.astype(BF).astype(F32)


def _edge2_body(ehb, ga, gp, gu, w1, pb1, w2, b2, eh_o):
    e_in = jnp.concatenate([ehb[...], ga[...], gp[...], gu[...]], axis=1)
    z = jax.nn.relu(_dot(e_in, w1[...]) + pb1[...])
    eh_o[...] = _dot(z, w2[...]) + b2[...]


def _node_upd_body(xhb, agge, urep, w1, bn1, w2, bn2, aggn_o):
    n_in = jnp.concatenate([xhb[...], agge[...], urep[...]], axis=1)
    h = jax.nn.relu(_dot(n_in, w1[...]) + bn1[...])
    x2 = _dot(h, w2[...]) + bn2[...]
    rows = lax.broadcasted_iota(jnp.int32, (GPB, NODE_BLK), 1)
    cols = lax.broadcasted_iota(jnp.int32, (GPB, NODE_BLK), 0)
    st = (rows // NPG == cols).astype(F32)
    aggn_o[...] = _sdot(st, x2)[None]


def _global_upd_body(uh, aggn, aggg, w1, bu1, w2, bu2,
                     hw1, hb1, hw2, hb2, q_o):
    u_in = jnp.concatenate([uh[...], aggn[...], aggg[...]], axis=1)
    h = jax.nn.relu(_dot(u_in, w1[...]) + bu1[...])
    u2 = _dot(h, w2[...]) + bu2[...]
    hh = jax.nn.relu(_dot(u2, hw1[...]) + hb1[...])
    q_o[...] = _dot(hh, hw2[...]) + hb2[...]


def _full(shape):
    n = len(shape)
    return pl.BlockSpec(shape, lambda *_: (0,) * n)


def _global_embed(u, w1, b1, w2, b2, w3, b3):
    return pl.pallas_call(
        _global_body,
        in_specs=[_full((B_PAD, 16)),
                  _full((16, 512)), _full((1, 512)),
                  _full((512, 512)), _full((1, 512)),
                  _full((512, 512)), _full((1, 512))],
        out_specs=[_full((B_PAD, 512)), _full((B_PAD, 512))],
        out_shape=[jax.ShapeDtypeStruct((B_PAD, 512), F32),
                   jax.ShapeDtypeStruct((B_PAD, 512), F32)],
    )(u, w1, b1, w2, b2, w3, b3)


def _node_embed(xc, w1, b1, w2, b2, w3, b3):
    return pl.pallas_call(
        _node_embed_body,
        grid=(N_NODE_BLKS,),
        in_specs=[pl.BlockSpec((NODE_BLK, 21), lambda i: (i, 0)),
                  _full((21, 512)), _full((1, 512)),
                  _full((512, 512)), _full((1, 512)),
                  _full((512, 512)), _full((1, 512))],
        out_specs=pl.BlockSpec((NODE_BLK, 512), lambda i: (i, 0)),
        out_shape=jax.ShapeDtypeStruct((NN_PAD, 512), F32),
    )(xc, w1, b1, w2, b2, w3, b3)


def _edge1(ea, w1, b1, w2, b2, w3, b3):
    return pl.pallas_call(
        _edge1_body,
        grid=(N_EDGE_BLKS,),
        in_specs=[pl.BlockSpec((EDGE_BLK, 4), lambda i: (i, 0)),
                  _full((4, 256)), _full((1, 256)),
                  _full((256, 256)), _full((1, 256)),
                  _full((256, 256)), _full((1, 256))],
        out_specs=pl.BlockSpec((EDGE_BLK, 256), lambda i: (i, 0)),
        out_shape=jax.ShapeDtypeStruct((NE_PAD, 256), F32),
    )(ea, w1, b1, w2, b2, w3, b3)


def _edge2(ehb, ga, gp, gu, w1, pb1, w2, b2):
    def espec(n):
        return pl.BlockSpec((EDGE_BLK, n), lambda i: (i, 0))
    return pl.pallas_call(
        _edge2_body,
        grid=(N_EDGE_BLKS,),
        in_specs=[espec(256), espec(512), espec(512), espec(512),
                  _full((1792, 256)), _full((1, 256)),
                  _full((256, 256)), _full((1, 256))],
        out_specs=espec(256),
        out_shape=jax.ShapeDtypeStruct((NE_PAD, 256), F32),
    )(ehb, ga, gp, gu, w1, pb1, w2, b2)


def _node_upd(xhb, agge, urep, w1, bn1, w2, bn2):
    return pl.pallas_call(
        _node_upd_body,
        grid=(N_NODE_BLKS,),
        in_specs=[pl.BlockSpec((NODE_BLK, 512), lambda i: (i, 0)),
                  pl.BlockSpec((NODE_BLK, 256), lambda i: (i, 0)),
                  pl.BlockSpec((NODE_BLK, 512), lambda i: (i, 0)),
                  _full((1280, 512)), _full((1, 512)),
                  _full((512, 512)), _full((1, 512))],
        out_specs=pl.BlockSpec((1, GPB, 512), lambda i: (i, 0, 0)),
        out_shape=jax.ShapeDtypeStruct((N_NODE_BLKS, GPB, 512), F32),
    )(xhb, agge, urep, w1, bn1, w2, bn2)


def _global_upd(uh, aggn, aggg, w1, bu1, w2, bu2, hw1, hb1, hw2, hb2):
    return pl.pallas_call(
        _global_upd_body,
        in_specs=[_full((B_PAD, 512)), _full((B_PAD, 512)),
                  _full((B_PAD, 256)),
                  _full((1280, 512)), _full((1, 512)),
                  _full((512, 512)), _full((1, 512)),
                  _full((512, 256)), _full((1, 256)),
                  _full((256, 1)), _full((1, 1))],
        out_specs=_full((B_PAD, 1)),
        out_shape=jax.ShapeDtypeStruct((B_PAD, 1), F32),
    )(uh, aggn, aggg, w1, bu1, w2, bu2, hw1, hb1, hw2, hb2)


def _sc_gather(xh_tab, u_tab, src2, dst2):
    mesh = plsc.VectorSubcoreMesh(core_axis_name="c", subcore_axis_name="s")
    cpw = NE_PAD // GCH // NW

    @functools.partial(
        pl.kernel, mesh=mesh,
        out_type=[jax.ShapeDtypeStruct((NE_PAD, 512), F32),
                  jax.ShapeDtypeStruct((NE_PAD, 512), F32),
                  jax.ShapeDtypeStruct((NE_PAD, 512), F32)],
        scratch_types=[pltpu.VMEM((1, GCH), jnp.int32),
                       pltpu.VMEM((1, GCH), jnp.int32),
                       pltpu.VMEM((1, GCH), jnp.int32),
                       pltpu.VMEM((GCH, 512), F32),
                       pltpu.VMEM((GCH, 512), F32),
                       pltpu.VMEM((GCH, 512), F32),
                       pltpu.SemaphoreType.DMA,
                       pltpu.SemaphoreType.DMA,
                       pltpu.SemaphoreType.DMA,
                       pltpu.SemaphoreType.DMA],
    )
    def k(xh_hbm, u_hbm, src_hbm, dst_hbm, ga_hbm, gp_hbm, gu_hbm,
          si_v, di_v, gi_v, ra_v, rp_v, ru_v, sga, sgp, sgu, semw):
        wid = lax.axis_index("s") * NC + lax.axis_index("c")

        def body(j, carry):
            chunk = j * NW + wid
            off = chunk * GCH
            pltpu.sync_copy(src_hbm.at[pl.ds(chunk, 1)], si_v)
            pltpu.sync_copy(dst_hbm.at[pl.ds(chunk, 1)], di_v)
            for kk in range(GCH // 16):
                sl = pl.ds(kk * 16, 16)
                gi_v[0, sl] = lax.div(si_v[0, sl], NPG)
            ha = pltpu.async_copy(xh_hbm.at[si_v.at[0]], ra_v, sga)
            hp = pltpu.async_copy(xh_hbm.at[di_v.at[0]], rp_v, sgp)
            hu = pltpu.async_copy(u_hbm.at[gi_v.at[0]], ru_v, sgu)
            ha.wait()
            wa = pltpu.async_copy(ra_v, ga_hbm.at[pl.ds(off, GCH)], semw)
            hp.wait()
            wp = pltpu.async_copy(rp_v, gp_hbm.at[pl.ds(off, GCH)], semw)
            hu.wait()
            wu = pltpu.async_copy(ru_v, gu_hbm.at[pl.ds(off, GCH)], semw)
            wa.wait()
            wp.wait()
            wu.wait()
            return carry

        lax.fori_loop(0, cpw, body, 0)

    return k(xh_tab, u_tab, src2, dst2)


def _sc_scatter(eh, src2, dst2):
    mesh = plsc.VectorSubcoreMesh(core_axis_name="c", subcore_axis_name="s")
    epb = NN_PAD // NS // CH
    gpb = B_PAD // NS

    @functools.partial(
        pl.kernel, mesh=mesh,
        out_type=[jax.ShapeDtypeStruct((NN_PAD, 256), F32),
                  jax.ShapeDtypeStruct((B_PAD, 256), F32)],
        scratch_types=[pltpu.VMEM((2, CH, 128), F32),
                       pltpu.VMEM((2, CH), jnp.int32),
                       pltpu.VMEM((2, CH), jnp.int32),
                       pltpu.VMEM((gpb, 128), F32),
                       pltpu.VMEM_SHARED((NN_PAD, 128), F32),
                       pltpu.VMEM_SHARED((B_PAD, 128), F32),
                       pltpu.SemaphoreType.DMA,
                       pltpu.SemaphoreType.DMA],
    )
    def k(eh_hbm, src_hbm, dst_hbm, agge_hbm, aggg_hbm,
          vb, di_v, gi_v, zbuf, acce, accg, sl0, sl1):
        c = lax.axis_index("c")
        s = lax.axis_index("s")
        base = s * SPS * CH
        brow = s * SPS

        def zrow(r, carry):
            for kk in range(128 // 16):
                zbuf[r, pl.ds(kk * 16, 16)] = jnp.zeros((16,), F32)
            return carry

        lax.fori_loop(0, gpb, zrow, 0)
        for kk in range(NN_PAD // NS // gpb):
            pltpu.sync_copy(zbuf, acce.at[pl.ds((s * 20 + kk) * gpb, gpb)])
        pltpu.sync_copy(zbuf, accg.at[pl.ds(s * gpb, gpb)])
        plsc.subcore_barrier()

        sls = (sl0, sl1)

        def body(jj, carry):
            hs = []
            for t in range(2):
                j = jj * 2 + t
                off = base + j * CH
                hs.append(pltpu.async_copy(
                    eh_hbm.at[pl.ds(off, CH), pl.ds(c * 128, 128)],
                    vb.at[t], sls[t]))
                pltpu.sync_copy(dst_hbm.at[brow + j], di_v.at[t])
                pltpu.sync_copy(src_hbm.at[brow + j], gi_v.at[t])
            for t in range(2):
                for kk in range(CH // 16):
                    sl = pl.ds(kk * 16, 16)
                    gi_v[t, sl] = lax.div(gi_v[t, sl], NPG)
                hs[t].wait()
                pltpu.sync_copy(vb.at[t], acce.at[di_v.at[t]], add=True)
                pltpu.sync_copy(vb.at[t], accg.at[gi_v.at[t]], add=True)
            return carry

        lax.fori_loop(0, SPS // 2, body, 0)
        plsc.subcore_barrier()

        for kk in range(epb):
            r = (s * epb + kk) * CH
            pltpu.sync_copy(acce.at[pl.ds(r, CH)],
                            agge_hbm.at[pl.ds(r, CH), pl.ds(c * 128, 128)])
        r2 = s * gpb
        pltpu.sync_copy(accg.at[pl.ds(r2, gpb)],
                        aggg_hbm.at[pl.ds(r2, gpb), pl.ds(c * 128, 128)])

    return k(eh, src2, dst2)


def _branch(p, xc, edge_attr, idx, u_pad):
    src2g, dst2g, src2s, dst2s = idx
    def w(name, i):
        return p[name][i]['W']

    def bb(name, i):
        return p[name][i]['b'].reshape(1, -1)

    peW1 = p['gnn']['phi_e'][0]['W']
    peb1 = p['gnn']['phi_e'][0]['b'].reshape(1, -1)
    peW2 = p['gnn']['phi_e'][1]['W']
    peb2 = p['gnn']['phi_e'][1]['b'].reshape(1, -1)
    pnW1 = p['gnn']['phi_n'][0]['W']
    pnb1 = p['gnn']['phi_n'][0]['b'].reshape(1, -1)
    pnW2 = p['gnn']['phi_n'][1]['W']
    pnb2 = p['gnn']['phi_n'][1]['b'].reshape(1, -1)
    puW1 = p['gnn']['phi_u'][0]['W']
    pub1 = p['gnn']['phi_u'][0]['b'].reshape(1, -1)
    puW2 = p['gnn']['phi_u'][1]['W']
    pub2 = p['gnn']['phi_u'][1]['b'].reshape(1, -1)
    hW1 = p['head'][0]['W']
    hb1 = bb('head', 0)
    hW2 = p['head'][1]['W']
    hb2 = bb('head', 1)

    uh, uhb = _global_embed(u_pad, w('g_emb', 0), bb('g_emb', 0),
                            w('g_emb', 1), bb('g_emb', 1),
                            w('g_emb', 2), bb('g_emb', 2))
    xhb = _node_embed(xc, w('n_emb', 0), bb('n_emb', 0),
                      w('n_emb', 1), bb('n_emb', 1),
                      w('n_emb', 2), bb('n_emb', 2))
    urep = jnp.repeat(uhb[:B], NPG, axis=0)
    ehb = _edge1(edge_attr, w('e_emb', 0), bb('e_emb', 0),
                 w('e_emb', 1), bb('e_emb', 1),
                 w('e_emb', 2), bb('e_emb', 2))
    ga, gp, gu = _sc_gather(xhb, uhb, src2g, dst2g)
    eh = _edge2(ehb, ga, gp, gu, peW1, peb1, peW2, peb2)
    agge, aggg = _sc_scatter(eh, src2s, dst2s)
    aggn = _node_upd(xhb, agge, urep, pnW1, pnb1, pnW2, pnb2)
    aggn_pad = jnp.pad(aggn.reshape(B, 512), ((0, B_PAD - B), (0, 0)))
    q = _global_upd(uh, aggn_pad, aggg, puW1, pub1, puW2, pub2,
                    hW1, hb1, hW2, hb2)
    return q[:B]


def kernel(x, edge_index, edge_attr, u, batch, action, params):
    del batch

    bsz, adim = action.shape
    apd = adim // 2
    robot = action.reshape(bsz, 2, apd)
    full = jnp.concatenate(
        [robot, jnp.zeros((bsz, NPG - 2, apd), dtype=x.dtype)], axis=1)
    xc = jnp.concatenate([x, full.reshape(-1, apd)], axis=1)
    npad = NE_PAD - N_EDGES
    src = jnp.concatenate(
        [edge_index[0], jnp.full((npad,), SRC_PAD, jnp.int32)])
    dst = jnp.concatenate(
        [edge_index[1], jnp.full((npad,), DST_PAD, jnp.int32)])
    edge_attr = jnp.pad(edge_attr, ((0, npad), (0, 0)))
    u_pad = jnp.pad(u, ((0, B_PAD - B), (0, 0)))
    idx = (src.reshape(NE_PAD // GCH, GCH), dst.reshape(NE_PAD // GCH, GCH),
           src.reshape(NE_PAD // CH, CH), dst.reshape(NE_PAD // CH, CH))

    q1 = _branch(params['branch1'], xc, edge_attr, idx, u_pad)
    q2 = _branch(params['branch2'], xc, edge_attr, idx, u_pad)
    return (q1, q2)
